# baseline (speedup 1.0000x reference)
import jax
import jax.numpy as jnp
from jax.experimental import pallas as pl
from jax.experimental.pallas import tpu as pltpu

ROWS = 100000
EMB = 128
CHUNK = 4000
NCHUNK = ROWS // CHUNK
SLOTS = 12
PREFETCH = 6


def _copy_body(u_in, i_in, u_out, i_out, vbuf, insem, outsem):
    streams = []
    for c in range(NCHUNK):
        sl = (pl.ds(c * CHUNK, CHUNK), slice(None))
        streams.append((u_in, u_out, sl))
        streams.append((i_in, i_out, sl))
    n = len(streams)

    def in_cp(k):
        t_in, _, sl = streams[k]
        s = k % SLOTS
        return pltpu.make_async_copy(t_in.at[sl], vbuf.at[s], insem.at[s])

    def out_cp(k):
        _, t_out, sl = streams[k]
        s = k % SLOTS
        return pltpu.make_async_copy(vbuf.at[s], t_out.at[sl], outsem.at[s])

    out_waited = [False] * n
    for k in range(min(PREFETCH, n)):
        in_cp(k).start()
    for k in range(n):
        pre = k + PREFETCH
        if pre < n:
            if pre >= SLOTS:
                out_cp(pre - SLOTS).wait()
                out_waited[pre - SLOTS] = True
            in_cp(pre).start()
        in_cp(k).wait()
        out_cp(k).start()
    for k in range(n):
        if not out_waited[k]:
            out_cp(k).wait()


def kernel(adj, user_emb, item_emb):
    del adj
    any_spec = pl.BlockSpec(memory_space=pl.ANY)
    out = pl.pallas_call(
        _copy_body,
        in_specs=[any_spec, any_spec],
        out_specs=[any_spec, any_spec],
        out_shape=[
            jax.ShapeDtypeStruct((ROWS, EMB), jnp.float32),
            jax.ShapeDtypeStruct((ROWS, EMB), jnp.float32),
        ],
        scratch_shapes=[
            pltpu.VMEM((SLOTS, CHUNK, EMB), jnp.float32),
            pltpu.SemaphoreType.DMA((SLOTS,)),
            pltpu.SemaphoreType.DMA((SLOTS,)),
        ],
    )(user_emb, item_emb)
    return (out[0], out[1])


# speedup vs baseline: 1.0442x; 1.0442x over previous
import jax
import jax.numpy as jnp
from jax.experimental import pallas as pl
from jax.experimental.pallas import tpu as pltpu

ROWS = 100000
EMB = 128
CHUNK = 8000
NCHUNK = ROWS // CHUNK
SLOTS = 8
PREFETCH = 4


def _copy_body(u_in, i_in, u_out, i_out, vbuf, insem, outsem):
    streams = []
    for c in range(NCHUNK):
        sl = (pl.ds(c * CHUNK, CHUNK), slice(None))
        streams.append((u_in, u_out, sl))
        streams.append((i_in, i_out, sl))
    n = len(streams)

    def in_cp(k):
        t_in, _, sl = streams[k]
        s = k % SLOTS
        return pltpu.make_async_copy(t_in.at[sl], vbuf.at[s], insem.at[s])

    def out_cp(k):
        _, t_out, sl = streams[k]
        s = k % SLOTS
        return pltpu.make_async_copy(vbuf.at[s], t_out.at[sl], outsem.at[s])

    out_waited = [False] * n
    for k in range(min(PREFETCH, n)):
        in_cp(k).start()
    for k in range(n):
        pre = k + PREFETCH
        if pre < n:
            if pre >= SLOTS:
                out_cp(pre - SLOTS).wait()
                out_waited[pre - SLOTS] = True
            in_cp(pre).start()
        in_cp(k).wait()
        out_cp(k).start()
    for k in range(n):
        if not out_waited[k]:
            out_cp(k).wait()


def kernel(adj, user_emb, item_emb):
    del adj
    any_spec = pl.BlockSpec(memory_space=pl.ANY)
    out = pl.pallas_call(
        _copy_body,
        in_specs=[any_spec, any_spec],
        out_specs=[any_spec, any_spec],
        out_shape=[
            jax.ShapeDtypeStruct((ROWS, EMB), jnp.float32),
            jax.ShapeDtypeStruct((ROWS, EMB), jnp.float32),
        ],
        scratch_shapes=[
            pltpu.VMEM((SLOTS, CHUNK, EMB), jnp.float32),
            pltpu.SemaphoreType.DMA((SLOTS,)),
            pltpu.SemaphoreType.DMA((SLOTS,)),
        ],
    )(user_emb, item_emb)
    return (out[0], out[1])
